# baseline (speedup 1.0000x reference)
import functools

import jax
import jax.numpy as jnp
from jax import lax
from jax.experimental import pallas as pl
from jax.experimental.pallas import tpu as pltpu
from jax.experimental.pallas import tpu_sc as plsc

BATCH = 16384
NUM_EMB = 1000000
HIST = 50
DIM = 64
LANES = 16
NUM_CORES = 2
NUM_SUBCORES = 16
NUM_WORKERS = NUM_CORES * NUM_SUBCORES
ROWS_PER_WORKER = BATCH // NUM_WORKERS
XCOLS = 128
GHIST = 56
CHUNK = 16
IDX_PER_CHUNK = CHUNK * GHIST
NUM_CHUNKS = ROWS_PER_WORKER // CHUNK
COMPACT_OFFS = (0, 16, 32, 40)

TW = 128
NBLK = NUM_EMB // TW
TAIL = NUM_EMB - NBLK * TW
NI = (NBLK + NUM_WORKERS - 1) // NUM_WORKERS


def _transpose_kernel(tabt_hbm, tail_hbm, out_hbm, in0, in1, ob0, ob1,
                      tail_b, si0, si1, so0, so1):
    wid = lax.axis_index("s") * NUM_CORES + lax.axis_index("c")
    ins, obs = (in0, in1), (ob0, ob1)
    sis, sos = (si0, si1), (so0, so1)
    row_ids = [lax.iota(jnp.int32, LANES) + j * LANES for j in range(4)]

    def blk_of(i):
        return jnp.minimum(i * NUM_WORKERS + wid, NBLK - 1)

    def load(i, b):
        return pltpu.make_async_copy(
            tabt_hbm.at[:, pl.ds(blk_of(i) * TW, TW)], ins[b], sis[b])

    def store(i, b):
        return pltpu.make_async_copy(
            obs[b], out_hbm.at[pl.ds(blk_of(i) * TW, TW)], sos[b])

    def transpose(src, dst, nrows):
        @pl.loop(0, nrows, step=4)
        def _(r0):
            for rr in range(4):
                r = r0 + rr
                col = jnp.full((LANES,), 0, jnp.int32) + r
                for j in range(4):
                    dst[r, pl.ds(j * LANES, LANES)] = plsc.load_gather(
                        src, [row_ids[j], col])

    load(0, 0).start()

    @pl.loop(0, NI + 1, step=2)
    def _(ch):
        for b in range(2):
            i = ch + b

            @pl.when(i < NI)
            def _():
                @pl.when(i + 1 < NI)
                def _():
                    load(i + 1, b ^ 1).start()

                load(i, b).wait()

                @pl.when(i >= 2)
                def _():
                    store(i - 2, b).wait()

                transpose(ins[b], obs[b], TW)
                store(i, b).start()

    store(NI - 2, (NI - 2) % 2).wait()
    store(NI - 1, (NI - 1) % 2).wait()

    @pl.when(wid == 0)
    def _():
        pltpu.sync_copy(tail_hbm, tail_b)
        transpose(tail_b, ob0, TAIL)
        pltpu.sync_copy(ob0.at[pl.ds(0, TAIL)],
                        out_hbm.at[pl.ds(NBLK * TW, TAIL)])


def _encoder_kernel(x_hbm, tab_hbm, out_hbm, xraw0, xraw1, idx0, idx1,
                    rows0, rows1, acc_v, sem0, sem1):
    wid = lax.axis_index("s") * NUM_CORES + lax.axis_index("c")
    base = wid * ROWS_PER_WORKER
    bufs = ((xraw0, idx0, rows0, sem0), (xraw1, idx1, rows1, sem1))

    def start_gather(ch, buf):
        xraw_v, idx_v, rows_v, sem = buf
        pltpu.sync_copy(x_hbm.at[pl.ds(base + ch * CHUNK, CHUNK)], xraw_v)
        for c in range(CHUNK):
            for off in COMPACT_OFFS:
                v = xraw_v[c, pl.ds(off, LANES)]
                idx_v[pl.ds(c * GHIST + off, LANES)] = v + v
        pltpu.async_copy(tab_hbm.at[idx_v], rows_v, sem)

    start_gather(0, bufs[0])

    @pl.loop(0, NUM_CHUNKS, step=2)
    def _(ch):
        for b in range(2):
            cur = ch + b
            _, idx_v, rows_v, sem = bufs[b]

            @pl.when(cur + 1 < NUM_CHUNKS)
            def _():
                start_gather(cur + 1, bufs[b ^ 1])

            pltpu.make_async_copy(tab_hbm.at[idx_v], rows_v, sem).wait()

            @pl.loop(0, CHUNK)
            def _(c):
                slices = [pl.ds(d * LANES, LANES) for d in range(DIM // LANES)]
                accs = [rows_v[c * GHIST, sl] for sl in slices]
                for l in range(1, HIST):
                    for d, sl in enumerate(slices):
                        accs[d] = accs[d] + rows_v[c * GHIST + l, sl]
                for d, sl in enumerate(slices):
                    acc_v[c, sl] = accs[d]

            pltpu.sync_copy(acc_v, out_hbm.at[pl.ds(base + cur * CHUNK, CHUNK)])


def kernel(x, table):
    mesh = plsc.VectorSubcoreMesh(core_axis_name="c", subcore_axis_name="s")

    trans = functools.partial(
        pl.kernel,
        out_type=jax.ShapeDtypeStruct((NUM_EMB, XCOLS), jnp.float32),
        mesh=mesh,
        scratch_types=[
            pltpu.VMEM((DIM, TW), jnp.float32),
            pltpu.VMEM((DIM, TW), jnp.float32),
            pltpu.VMEM((TW, XCOLS), jnp.float32),
            pltpu.VMEM((TW, XCOLS), jnp.float32),
            pltpu.VMEM((DIM, TW), jnp.float32),
            pltpu.SemaphoreType.DMA,
            pltpu.SemaphoreType.DMA,
            pltpu.SemaphoreType.DMA,
            pltpu.SemaphoreType.DMA,
        ],
        compiler_params=pltpu.CompilerParams(
            use_tc_tiling_on_sc=True, needs_layout_passes=False),
    )(_transpose_kernel)

    run = functools.partial(
        pl.kernel,
        out_type=jax.ShapeDtypeStruct((BATCH, DIM), jnp.float32),
        mesh=mesh,
        scratch_types=[
            pltpu.VMEM((CHUNK, XCOLS), jnp.int32),
            pltpu.VMEM((CHUNK, XCOLS), jnp.int32),
            pltpu.VMEM((IDX_PER_CHUNK,), jnp.int32),
            pltpu.VMEM((IDX_PER_CHUNK,), jnp.int32),
            pltpu.VMEM((IDX_PER_CHUNK, DIM), jnp.float32),
            pltpu.VMEM((IDX_PER_CHUNK, DIM), jnp.float32),
            pltpu.VMEM((CHUNK, DIM), jnp.float32),
            pltpu.SemaphoreType.DMA,
            pltpu.SemaphoreType.DMA,
        ],
        compiler_params=pltpu.CompilerParams(use_tc_tiling_on_sc=False),
    )(_encoder_kernel)

    xf = jnp.pad(x.astype(jnp.int32), ((0, 0), (0, XCOLS - HIST)),
                 mode="edge")
    tabt = table.T
    tail = jnp.pad(tabt[:, NBLK * TW:], ((0, 0), (0, TW - TAIL)))
    tp = trans(tabt, tail)
    tp2 = tp.reshape(2 * NUM_EMB, DIM)
    return run(xf, tp2)


# speedup vs baseline: 1.8876x; 1.8876x over previous
import functools

import jax
import jax.numpy as jnp
from jax import lax
from jax.experimental import pallas as pl
from jax.experimental.pallas import tpu as pltpu
from jax.experimental.pallas import tpu_sc as plsc

BATCH = 16384
NUM_EMB = 1000000
HIST = 50
DIM = 64
LANES = 16
NUM_CORES = 2
NUM_SUBCORES = 16
NUM_WORKERS = NUM_CORES * NUM_SUBCORES
ROWS_PER_WORKER = BATCH // NUM_WORKERS
XCOLS = 128
GHIST = 56
CHUNK = 16
IDX_PER_CHUNK = CHUNK * GHIST
NUM_CHUNKS = ROWS_PER_WORKER // CHUNK
COMPACT_OFFS = (0, 16, 32, 40)

TBW = 1024


def _encoder_kernel(x_hbm, tab_hbm, out_hbm, xraw0, xraw1, idx0, idx1,
                    rows0, rows1, acc_v, sem0, sem1):
    wid = lax.axis_index("s") * NUM_CORES + lax.axis_index("c")
    base = wid * ROWS_PER_WORKER
    bufs = ((xraw0, idx0, rows0, sem0), (xraw1, idx1, rows1, sem1))

    def start_gather(ch, buf):
        xraw_v, idx_v, rows_v, sem = buf
        pltpu.sync_copy(x_hbm.at[pl.ds(base + ch * CHUNK, CHUNK)], xraw_v)
        for c in range(CHUNK):
            for off in COMPACT_OFFS:
                v = xraw_v[c, pl.ds(off, LANES)]
                idx_v[pl.ds(c * GHIST + off, LANES)] = v + v
        pltpu.async_copy(tab_hbm.at[idx_v], rows_v, sem)

    start_gather(0, bufs[0])

    @pl.loop(0, NUM_CHUNKS, step=2)
    def _(ch):
        for b in range(2):
            cur = ch + b
            _, idx_v, rows_v, sem = bufs[b]

            @pl.when(cur + 1 < NUM_CHUNKS)
            def _():
                start_gather(cur + 1, bufs[b ^ 1])

            pltpu.make_async_copy(tab_hbm.at[idx_v], rows_v, sem).wait()

            @pl.loop(0, CHUNK)
            def _(c):
                slices = [pl.ds(d * LANES, LANES) for d in range(DIM // LANES)]
                accs = [rows_v[c * GHIST, sl] for sl in slices]
                for l in range(1, HIST):
                    for d, sl in enumerate(slices):
                        accs[d] = accs[d] + rows_v[c * GHIST + l, sl]
                for d, sl in enumerate(slices):
                    acc_v[c, sl] = accs[d]

            pltpu.sync_copy(acc_v, out_hbm.at[pl.ds(base + cur * CHUNK, CHUNK)])


def _tc_transpose_body(in_ref, out_ref):
    out_ref[:, :DIM] = in_ref[...].T


def kernel(x, table):
    mesh = plsc.VectorSubcoreMesh(core_axis_name="c", subcore_axis_name="s")

    trans = pl.pallas_call(
        _tc_transpose_body,
        grid=((NUM_EMB + TBW - 1) // TBW,),
        in_specs=[pl.BlockSpec((DIM, TBW), lambda g: (0, g))],
        out_specs=pl.BlockSpec((TBW, XCOLS), lambda g: (g, 0)),
        out_shape=jax.ShapeDtypeStruct((NUM_EMB, XCOLS), jnp.float32),
    )

    run = functools.partial(
        pl.kernel,
        out_type=jax.ShapeDtypeStruct((BATCH, DIM), jnp.float32),
        mesh=mesh,
        scratch_types=[
            pltpu.VMEM((CHUNK, XCOLS), jnp.int32),
            pltpu.VMEM((CHUNK, XCOLS), jnp.int32),
            pltpu.VMEM((IDX_PER_CHUNK,), jnp.int32),
            pltpu.VMEM((IDX_PER_CHUNK,), jnp.int32),
            pltpu.VMEM((IDX_PER_CHUNK, DIM), jnp.float32),
            pltpu.VMEM((IDX_PER_CHUNK, DIM), jnp.float32),
            pltpu.VMEM((CHUNK, DIM), jnp.float32),
            pltpu.SemaphoreType.DMA,
            pltpu.SemaphoreType.DMA,
        ],
        compiler_params=pltpu.CompilerParams(use_tc_tiling_on_sc=False),
    )(_encoder_kernel)

    xf = jnp.pad(x.astype(jnp.int32), ((0, 0), (0, XCOLS - HIST)),
                 mode="edge")
    tp = trans(table.T)
    tp2 = tp.reshape(2 * NUM_EMB, DIM)
    return run(xf, tp2)
